# baseline (speedup 1.0000x reference)
import functools

import jax
import jax.numpy as jnp
from jax import lax
from jax.experimental import pallas as pl
from jax.experimental.pallas import tpu as pltpu
from jax.experimental.pallas import tpu_sc as plsc

_NUM_CORES = 2
_NUM_SUBCORES = 16
_NW = _NUM_CORES * _NUM_SUBCORES
_LANES = 16
_CHUNK = 24
_UNROLL = 8


@functools.partial(jax.jit, static_argnums=(3, 4, 5, 6, 7))
def _embed_add(ids_pad, token_embedding, ctx_flat, B, W, WP, V, D):
    rows_per_w = B // _NW
    n_main = W // _CHUNK
    tail = W - n_main * _CHUNK
    mesh = plsc.VectorSubcoreMesh(core_axis_name="c", subcore_axis_name="s")

    @functools.partial(
        pl.kernel,
        out_type=jax.ShapeDtypeStruct((B, W, D), jnp.float32),
        mesh=mesh,
        scratch_types=[
            pltpu.VMEM((rows_per_w * WP,), jnp.int32),
            pltpu.VMEM((_CHUNK, D), jnp.float32),
            pltpu.VMEM((_CHUNK, D), jnp.float32),
            pltpu.VMEM((tail, D), jnp.float32),
            pltpu.VMEM((tail, D), jnp.float32),
            pltpu.VMEM((_CHUNK * D,), jnp.float32),
            pltpu.SemaphoreType.DMA,
            pltpu.SemaphoreType.DMA,
            pltpu.SemaphoreType.DMA,
            pltpu.SemaphoreType.DMA,
        ],
    )
    def k(ids_hbm, table_hbm, ctx_hbm, out_hbm,
          idx_v, rows0, rows1, trows0, trows1, ctx_c,
          gsem0, gsem1, wsem0, wsem1):
        wid = lax.axis_index("s") * _NUM_CORES + lax.axis_index("c")
        base = wid * rows_per_w
        pltpu.sync_copy(ids_hbm.at[pl.ds(base * WP, rows_per_w * WP)], idx_v)

        def run_chunk(off, cw, rbufs, gsems, wsems):
            pltpu.sync_copy(ctx_hbm.at[pl.ds(off * D, cw * D)],
                            ctx_c.at[pl.ds(0, cw * D)])

            def gsrc(r):
                return table_hbm.at[idx_v.at[pl.ds(r * WP + off, cw)]]

            def wdst(r):
                return out_hbm.at[base + r, pl.ds(off, cw)]

            def add_ctx(buf):
                def body(w, carry):
                    cbase = w * D
                    for j in range(D // _LANES):
                        plsc.addupdate(
                            buf.at[w, pl.ds(j * _LANES, _LANES)],
                            ctx_c[pl.ds(cbase + j * _LANES, _LANES)])
                    return carry

                lax.fori_loop(0, cw, body, 0)

            pltpu.async_copy(gsrc(0), rbufs[0], gsems[0])

            @pl.loop(0, rows_per_w, step=2)
            def _(r):
                pltpu.make_async_copy(gsrc(r), rbufs[0], gsems[0]).wait()

                @pl.when(r > 0)
                def _():
                    pltpu.make_async_copy(rbufs[1], wdst(r - 1), wsems[1]).wait()

                pltpu.async_copy(gsrc(r + 1), rbufs[1], gsems[1])
                add_ctx(rbufs[0])
                pltpu.async_copy(rbufs[0], wdst(r), wsems[0])

                pltpu.make_async_copy(gsrc(r + 1), rbufs[1], gsems[1]).wait()
                pltpu.make_async_copy(rbufs[0], wdst(r), wsems[0]).wait()

                @pl.when(r + 2 < rows_per_w)
                def _():
                    pltpu.async_copy(gsrc(r + 2), rbufs[0], gsems[0])

                add_ctx(rbufs[1])
                pltpu.async_copy(rbufs[1], wdst(r + 1), wsems[1])

            pltpu.make_async_copy(rbufs[1], wdst(rows_per_w - 1),
                                  wsems[1]).wait()

        for c in range(n_main):
            run_chunk(c * _CHUNK, _CHUNK, (rows0, rows1),
                      (gsem0, gsem1), (wsem0, wsem1))
        run_chunk(n_main * _CHUNK, tail, (trows0, trows1),
                  (gsem0, gsem1), (wsem0, wsem1))

    return k(ids_pad, token_embedding, ctx_flat)


def kernel(prompt_ids, token_embedding, ctx):
    B, W = prompt_ids.shape
    V, D = token_embedding.shape
    WP = (W + 7) // 8 * 8
    ids = prompt_ids.astype(jnp.int32)
    ids_pad = jnp.pad(ids, ((0, 0), (0, WP - W))).reshape(-1)
    return _embed_add(ids_pad, token_embedding, ctx.reshape(-1), B, W, WP, V, D)
